# baseline (speedup 1.0000x reference)
import jax
import jax.numpy as jnp
from jax.experimental import pallas as pl
from jax.experimental.pallas import tpu as pltpu

CLS = 1000
N0 = 4096
N1 = 26
BC = 1000
NCB = CLS // BC


def _onehot_body(x_ref, o_ref):
    i1 = pl.program_id(0)
    jc = pl.program_id(1)
    xrow = x_ref[pl.ds(i1, 1), :]
    ci = jax.lax.broadcasted_iota(jnp.int32, (BC, N0), 0) + jc * BC
    o_ref[0] = (ci == xrow).astype(jnp.float32)


def kernel(x):
    xt = x.T
    out_t = pl.pallas_call(
        _onehot_body,
        grid=(N1, NCB),
        in_specs=[pl.BlockSpec((N1, N0), lambda i, j: (0, 0))],
        out_specs=pl.BlockSpec((1, BC, N0), lambda i, j: (i, j, 0)),
        out_shape=jax.ShapeDtypeStruct((N1, CLS, N0), jnp.float32),
        compiler_params=pltpu.CompilerParams(
            dimension_semantics=("arbitrary", "arbitrary"),
        ),
    )(xt)
    return jnp.transpose(out_t, (2, 0, 1))
